# baseline (speedup 1.0000x reference)
import jax
import jax.numpy as jnp
from jax import lax
from jax.experimental import pallas as pl
from jax.experimental.pallas import tpu as pltpu
from jax.experimental.pallas import tpu_sc as plsc

NC, NS = 2, 16
NW = NC * NS
CHUNK = 128
TBLK = 2048


def _eye(n):
    return (lax.broadcasted_iota(jnp.int32, (n, n), 0) ==
            lax.broadcasted_iota(jnp.int32, (n, n), 1)).astype(jnp.float32)


def _table_body(t1_ref, t2_ref, out_ref):
    xx = jnp.concatenate([t1_ref[...], t2_ref[...]], axis=0)
    hi = xx.astype(jnp.bfloat16)
    lo = (xx - hi.astype(jnp.float32)).astype(jnp.bfloat16)
    e = _eye(128).astype(jnp.bfloat16)
    dn = (((0,), (0,)), ((), ()))
    out_ref[...] = (
        lax.dot_general(hi, e, dn, preferred_element_type=jnp.float32) +
        lax.dot_general(lo, e, dn, preferred_element_type=jnp.float32))


def _transpose_table(tableT):
    D, V = tableT.shape
    grid = (V + TBLK - 1) // TBLK
    h = TBLK // 2
    return pl.pallas_call(
        _table_body,
        grid=(grid,),
        in_specs=[pl.BlockSpec((D, h), lambda i: (0, 2 * i)),
                  pl.BlockSpec(
                      (D, h),
                      lambda i: (0, jnp.minimum(2 * i + 1, (V - 1) // h))),
                  ],
        out_specs=pl.BlockSpec((h, 128), lambda i: (i, 0)),
        out_shape=jax.ShapeDtypeStruct((grid * h, 128), jnp.float32),
    )(tableT, tableT)


def _out_body(g_ref, out_ref):
    g = g_ref[...].reshape(128, 100, 128)
    out3 = lax.dot_general(g, _eye(128), (((0,), (0,)), ((), ())),
                           preferred_element_type=jnp.float32)
    out_ref[...] = out3.reshape(200, 8, 1, 8, 128)


def _transpose_out(g):
    return pl.pallas_call(
        _out_body,
        grid=(32,),
        in_specs=[pl.BlockSpec((12800, 128), lambda tc: (tc, 0))],
        out_specs=pl.BlockSpec((200, 8, 1, 8, 128),
                               lambda tc: (0, 0, tc, 0, 0)),
        out_shape=jax.ShapeDtypeStruct((200, 8, 32, 8, 128), jnp.float32),
        compiler_params=pltpu.CompilerParams(
            vmem_limit_bytes=100 * 1024 * 1024),
    )(g)


def kernel(x, table):
    B, H = x.shape
    V, D = table.shape
    N = B * H
    n_chunks = N // CHUNK
    per_w = n_chunks // NW
    assert n_chunks * CHUNK == N and per_w * NW == n_chunks

    table_p = _transpose_table(jnp.transpose(table, (1, 0)))
    V2 = table_p.shape[0] * 2
    table_rm = table_p.reshape(V2, D)

    xi = x.reshape(n_chunks, CHUNK).astype(jnp.int32)
    xf = ((xi & ~jnp.int32(2047)) + ((xi & 1023) << 1) + ((xi >> 10) & 1))
    mesh = plsc.VectorSubcoreMesh(
        core_axis_name="c", subcore_axis_name="s",
        num_cores=NC, num_subcores=NS,
    )

    NBUF = 8
    KMAX = per_w // NBUF
    assert KMAX * NBUF == per_w

    def body(idx_hbm, table_hbm, out_hbm, idx_v, rows_v, *sems):
        gsem, wsem = sems[:NBUF], sems[NBUF:]
        wid = lax.axis_index("s") * NC + lax.axis_index("c")
        cbase = wid * per_w
        pltpu.sync_copy(idx_hbm.at[pl.ds(cbase, per_w)], idx_v)

        for b in range(NBUF):
            pltpu.async_copy(
                table_hbm.at[idx_v.at[b]], rows_v.at[b], gsem[b])

        def step(k, carry):
            for b in range(NBUF):
                j = k * NBUF + b
                pltpu.make_async_copy(
                    table_hbm.at[idx_v.at[0]], rows_v.at[b], gsem[b]).wait()
                pltpu.async_copy(
                    rows_v.at[b],
                    out_hbm.at[pl.ds((cbase + j) * CHUNK, CHUNK)], wsem[b])
            for b in range(NBUF):
                pltpu.make_async_copy(
                    rows_v.at[b], out_hbm.at[pl.ds(0, CHUNK)], wsem[b]).wait()

                @pl.when(k < KMAX - 1)
                def _():
                    pltpu.async_copy(
                        table_hbm.at[idx_v.at[(k + 1) * NBUF + b]],
                        rows_v.at[b], gsem[b])
            return carry

        lax.fori_loop(0, KMAX, step, 0)

    g = pl.kernel(
        body,
        out_type=jax.ShapeDtypeStruct((N, D), jnp.float32),
        mesh=mesh,
        compiler_params=pltpu.CompilerParams(use_tc_tiling_on_sc=False),
        scratch_types=[
            pltpu.VMEM((per_w, CHUNK), jnp.int32),
            pltpu.VMEM((NBUF, CHUNK, D), jnp.float32),
        ] + [pltpu.SemaphoreType.DMA] * (2 * NBUF),
    )(xf, table_rm)

    out4 = _transpose_out(g.reshape(N * D // 128, 128))
    return out4.transpose(2, 4, 0, 1, 3).reshape(B, H, D)


# speedup vs baseline: 1.2457x; 1.2457x over previous
import jax
import jax.numpy as jnp
from jax import lax
from jax.experimental import pallas as pl
from jax.experimental.pallas import tpu as pltpu
from jax.experimental.pallas import tpu_sc as plsc

NC, NS = 2, 16
NW = NC * NS
CHUNK = 128
TBLK = 4096


def _eye(n):
    return (lax.broadcasted_iota(jnp.int32, (n, n), 0) ==
            lax.broadcasted_iota(jnp.int32, (n, n), 1)).astype(jnp.float32)


def _table_body(t1_ref, t2_ref, out_ref):
    xx = jnp.concatenate([t1_ref[...], t2_ref[...]], axis=0)
    out_ref[...] = lax.dot_general(
        xx, _eye(128), (((0,), (0,)), ((), ())),
        preferred_element_type=jnp.float32)


def _transpose_table(tableT):
    D, V = tableT.shape
    grid = (V + TBLK - 1) // TBLK
    h = TBLK // 2
    return pl.pallas_call(
        _table_body,
        grid=(grid,),
        in_specs=[pl.BlockSpec((D, h), lambda i: (0, 2 * i)),
                  pl.BlockSpec(
                      (D, h),
                      lambda i: (0, jnp.minimum(2 * i + 1, (V - 1) // h))),
                  ],
        out_specs=pl.BlockSpec((h, 128), lambda i: (i, 0)),
        out_shape=jax.ShapeDtypeStruct((grid * h, 128), jnp.float32),
    )(tableT, tableT)


def _out_body(g_ref, out_ref):
    g = g_ref[...].reshape(128, 100, 128)
    out3 = lax.dot_general(g, _eye(128), (((0,), (0,)), ((), ())),
                           preferred_element_type=jnp.float32)
    out_ref[...] = out3.reshape(200, 8, 1, 8, 128)


def _transpose_out(g):
    return pl.pallas_call(
        _out_body,
        grid=(32,),
        in_specs=[pl.BlockSpec((12800, 128), lambda tc: (tc, 0))],
        out_specs=pl.BlockSpec((200, 8, 1, 8, 128),
                               lambda tc: (0, 0, tc, 0, 0)),
        out_shape=jax.ShapeDtypeStruct((200, 8, 32, 8, 128), jnp.float32),
        compiler_params=pltpu.CompilerParams(
            vmem_limit_bytes=100 * 1024 * 1024),
    )(g)


def kernel(x, table):
    B, H = x.shape
    V, D = table.shape
    N = B * H
    n_chunks = N // CHUNK
    per_w = n_chunks // NW
    assert n_chunks * CHUNK == N and per_w * NW == n_chunks

    table_p = _transpose_table(jnp.transpose(table, (1, 0)))
    V2 = table_p.shape[0] * 2
    table_rm = table_p.reshape(V2, D)

    xi = x.reshape(n_chunks, CHUNK).astype(jnp.int32)
    hm = TBLK // 2 - 1
    sh = (TBLK // 2).bit_length() - 1
    xf = ((xi & ~jnp.int32(TBLK - 1)) + ((xi & hm) << 1) + ((xi >> sh) & 1))
    mesh = plsc.VectorSubcoreMesh(
        core_axis_name="c", subcore_axis_name="s",
        num_cores=NC, num_subcores=NS,
    )

    NBUF = 8
    KMAX = per_w // NBUF
    assert KMAX * NBUF == per_w

    def body(idx_hbm, table_hbm, out_hbm, idx_v, rows_v, *sems):
        gsem, wsem = sems[:NBUF], sems[NBUF:]
        wid = lax.axis_index("s") * NC + lax.axis_index("c")
        cbase = wid * per_w
        pltpu.sync_copy(idx_hbm.at[pl.ds(cbase, per_w)], idx_v)

        for b in range(NBUF):
            pltpu.async_copy(
                table_hbm.at[idx_v.at[b]], rows_v.at[b], gsem[b])

        def step(k, carry):
            for b in range(NBUF):
                j = k * NBUF + b
                pltpu.make_async_copy(
                    table_hbm.at[idx_v.at[0]], rows_v.at[b], gsem[b]).wait()
                pltpu.async_copy(
                    rows_v.at[b],
                    out_hbm.at[pl.ds((cbase + j) * CHUNK, CHUNK)], wsem[b])
            for b in range(NBUF):
                pltpu.make_async_copy(
                    rows_v.at[b], out_hbm.at[pl.ds(0, CHUNK)], wsem[b]).wait()

                @pl.when(k < KMAX - 1)
                def _():
                    pltpu.async_copy(
                        table_hbm.at[idx_v.at[(k + 1) * NBUF + b]],
                        rows_v.at[b], gsem[b])
            return carry

        lax.fori_loop(0, KMAX, step, 0)

    g = pl.kernel(
        body,
        out_type=jax.ShapeDtypeStruct((N, D), jnp.float32),
        mesh=mesh,
        compiler_params=pltpu.CompilerParams(use_tc_tiling_on_sc=False),
        scratch_types=[
            pltpu.VMEM((per_w, CHUNK), jnp.int32),
            pltpu.VMEM((NBUF, CHUNK, D), jnp.float32),
        ] + [pltpu.SemaphoreType.DMA] * (2 * NBUF),
    )(xf, table_rm)

    out4 = _transpose_out(g.reshape(N * D // 128, 128))
    return out4.transpose(2, 4, 0, 1, 3).reshape(B, H, D)
